# Initial kernel scaffold; baseline (speedup 1.0000x reference)
import jax
import jax.numpy as jnp
from jax.experimental import pallas as pl


def kernel(x, edge_index, W1, b1, W2, b2, W3, b3, W4, b4):
    raise NotImplementedError("write your pallas kernel here")


# speedup vs baseline: 14.4680x; 14.4680x over previous
import functools

import jax
import jax.numpy as jnp
from jax import lax
from jax.experimental import pallas as pl
from jax.experimental.pallas import tpu as pltpu
from jax.experimental.pallas import tpu_sc as plsc

N = 10000
E = 320000
EC = 128
NC = 2
NT = 16
R = 2560
EPAD = R * EC
NPAD = NT * 640
SL = NPAD // NT

_mesh = plsc.VectorSubcoreMesh(core_axis_name="c", subcore_axis_name="s")
_sc_params = pltpu.CompilerParams(use_tc_tiling_on_sc=False)


def _deg_body(dst2d_hbm, degp_hbm, acc, zbuf, didx, ones):
  c = lax.axis_index("c")
  s = lax.axis_index("s")
  w = s * NC + c
  nrows = R // (NC * NT)

  def zb(i, _):
    zbuf[pl.ds(i * 16, 16)] = jnp.zeros((16,), jnp.float32)
    return 0
  lax.fori_loop(0, SL // 16, zb, 0)

  def ob(i, _):
    ones[pl.ds(i * 16, 16)] = jnp.ones((16,), jnp.float32)
    return 0
  lax.fori_loop(0, EC // 16, ob, 0)

  pltpu.sync_copy(dst2d_hbm.at[pl.ds(w * nrows, nrows), :], didx)
  pltpu.sync_copy(zbuf, acc.at[pl.ds(s * SL, SL)])
  plsc.subcore_barrier()

  def body(j, _):
    pltpu.sync_copy(ones, acc.at[didx.at[j]], add=True)
    return 0
  lax.fori_loop(0, nrows, body, 0)

  plsc.subcore_barrier()
  pltpu.sync_copy(acc.at[pl.ds(s * SL, SL)],
                  degp_hbm.at[c, 0, pl.ds(s * SL, SL)])


_deg_kernel = pl.kernel(
    _deg_body,
    out_type=jax.ShapeDtypeStruct((NC, 1, NPAD), jnp.float32),
    mesh=_mesh,
    compiler_params=_sc_params,
    scratch_types=[
        pltpu.VMEM_SHARED((NPAD,), jnp.float32),
        pltpu.VMEM((SL,), jnp.float32),
        pltpu.VMEM((R // (NC * NT), EC), jnp.int32),
        pltpu.VMEM((EC,), jnp.float32),
    ],
)


def _scatter_body(y2_hbm, idx3_hbm, dst2d_hbm, out_hbm,
                  acc, sidx, didx, rows0, rows1, g0, g1):
  c = lax.axis_index("c")
  s = lax.axis_index("s")
  nrows = R // NT
  base = s * nrows

  pltpu.sync_copy(idx3_hbm.at[c, pl.ds(base, nrows), :], sidx)
  pltpu.sync_copy(dst2d_hbm.at[pl.ds(base, nrows), :], didx)
  pltpu.sync_copy(y2_hbm.at[pl.ds(c * NPAD + s * SL, SL), :],
                  acc.at[pl.ds(s * SL, SL), :])
  plsc.subcore_barrier()

  def gstart(j, rbuf, sem):
    pltpu.async_copy(y2_hbm.at[sidx.at[j]], rbuf, sem)

  def gwait(j, rbuf, sem):
    pltpu.make_async_copy(y2_hbm.at[sidx.at[j]], rbuf, sem).wait()

  def scat(j, rbuf):
    pltpu.sync_copy(rbuf, acc.at[didx.at[j]], add=True)

  gstart(0, rows0, g0)
  gstart(1, rows1, g1)
  nhalf = nrows // 2

  def body(j2, _):
    j = 2 * j2
    gwait(j, rows0, g0)
    scat(j, rows0)

    @pl.when(j2 + 1 < nhalf)
    def _():
      gstart(j + 2, rows0, g0)

    gwait(j + 1, rows1, g1)
    scat(j + 1, rows1)

    @pl.when(j2 + 1 < nhalf)
    def _():
      gstart(j + 3, rows1, g1)
    return 0
  lax.fori_loop(0, nhalf, body, 0)

  plsc.subcore_barrier()
  pltpu.sync_copy(acc.at[pl.ds(s * SL, SL), :],
                  out_hbm.at[c, pl.ds(s * SL, SL), :])


@functools.cache
def _scatter_kernel(dh):
  return pl.kernel(
      _scatter_body,
      out_type=jax.ShapeDtypeStruct((NC, NPAD, dh), jnp.float32),
      mesh=_mesh,
      compiler_params=_sc_params,
      scratch_types=[
          pltpu.VMEM_SHARED((NPAD, dh), jnp.float32),
          pltpu.VMEM((R // NT, EC), jnp.int32),
          pltpu.VMEM((R // NT, EC), jnp.int32),
          pltpu.VMEM((EC, dh), jnp.float32),
          pltpu.VMEM((EC, dh), jnp.float32),
          pltpu.SemaphoreType.DMA,
          pltpu.SemaphoreType.DMA,
      ],
  )


BR = 1000


def _mm1_body(x_ref, d0_ref, d1_ref, w_ref, oL_ref, oR_ref):
  dinv = lax.rsqrt(d0_ref[...] + d1_ref[...] + 1.0)
  y = jnp.dot(x_ref[...], w_ref[...],
              preferred_element_type=jnp.float32) * dinv
  h = y.shape[1] // 2
  oL_ref[...] = y[:, :h]
  oR_ref[...] = y[:, h:]


def _mid_body(aL_ref, aR_ref, d0_ref, d1_ref, bL_ref, bR_ref, wt_ref, wb_ref,
              oL_ref, oR_ref):
  dinv = lax.rsqrt(d0_ref[...] + d1_ref[...] + 1.0)
  hL = jnp.maximum(aL_ref[...] * dinv + bL_ref[...], 0.0)
  hR = jnp.maximum(aR_ref[...] * dinv + bR_ref[...], 0.0)
  y = (jnp.dot(hL, wt_ref[...], preferred_element_type=jnp.float32)
       + jnp.dot(hR, wb_ref[...], preferred_element_type=jnp.float32)) * dinv
  h = y.shape[1] // 2
  oL_ref[...] = y[:, :h]
  oR_ref[...] = y[:, h:]


def _fin_body(aL_ref, aR_ref, d0_ref, d1_ref, bL_ref, bR_ref, o_ref):
  dinv = lax.rsqrt(d0_ref[...] + d1_ref[...] + 1.0)
  o_ref[...] = jnp.concatenate(
      [aL_ref[...] * dinv + bL_ref[...], aR_ref[...] * dinv + bR_ref[...]],
      axis=1)


def _row_spec(width):
  return pl.BlockSpec((BR, width), lambda i: (i, 0))


def _whole_spec(shape):
  return pl.BlockSpec(shape, lambda i: tuple(0 for _ in shape))


def _mm1(x, d0, d1, w):
  fout = w.shape[1]
  return pl.pallas_call(
      _mm1_body,
      grid=(N // BR,),
      in_specs=[_row_spec(x.shape[1]), _row_spec(1), _row_spec(1),
                _whole_spec(w.shape)],
      out_specs=[_row_spec(fout // 2), _row_spec(fout // 2)],
      out_shape=[jax.ShapeDtypeStruct((N, fout // 2), jnp.float32)] * 2,
  )(x, d0, d1, w)


def _mid(aL, aR, d0, d1, bL, bR, wt, wb):
  dh = aL.shape[1]
  fout = wt.shape[1]
  return pl.pallas_call(
      _mid_body,
      grid=(N // BR,),
      in_specs=[_row_spec(dh), _row_spec(dh), _row_spec(1), _row_spec(1),
                _whole_spec(bL.shape), _whole_spec(bR.shape),
                _whole_spec(wt.shape), _whole_spec(wb.shape)],
      out_specs=[_row_spec(fout // 2), _row_spec(fout // 2)],
      out_shape=[jax.ShapeDtypeStruct((N, fout // 2), jnp.float32)] * 2,
  )(aL, aR, d0, d1, bL, bR, wt, wb)


def _fin(aL, aR, d0, d1, bL, bR):
  dh = aL.shape[1]
  return pl.pallas_call(
      _fin_body,
      grid=(N // BR,),
      in_specs=[_row_spec(dh), _row_spec(dh), _row_spec(1), _row_spec(1),
                _whole_spec(bL.shape), _whole_spec(bR.shape)],
      out_specs=_row_spec(2 * dh),
      out_shape=jax.ShapeDtypeStruct((N, 2 * dh), jnp.float32),
  )(aL, aR, d0, d1, bL, bR)


def _pack(yL, yR):
  z = jnp.zeros((NPAD - N, yL.shape[1]), jnp.float32)
  return jnp.concatenate([yL, z, yR, z], axis=0)


@jax.jit
def _run(x, edge_index, W1, b1, W2, b2, W3, b3, W4, b4):
  src = edge_index[0]
  dst = edge_index[1]
  src_p = jnp.concatenate([src, jnp.zeros((EPAD - E,), jnp.int32)])
  dst_p = jnp.concatenate([dst, jnp.full((EPAD - E,), N, jnp.int32)])
  src2d = src_p.reshape(R, EC)
  dst2d = dst_p.reshape(R, EC)
  idx3 = jnp.stack([src2d, src2d + NPAD])

  degp = _deg_kernel(dst2d)
  d0 = degp[0, 0, :N, None]
  d1 = degp[1, 0, :N, None]

  yL, yR = _mm1(x, d0, d1, W1)
  a = _scatter_kernel(32)(_pack(yL, yR), idx3, dst2d)
  yL, yR = _mid(a[0, :N], a[1, :N], d0, d1, b1[None, :32], b1[None, 32:],
                W2[:32], W2[32:])
  a = _scatter_kernel(16)(_pack(yL, yR), idx3, dst2d)
  yL, yR = _mid(a[0, :N], a[1, :N], d0, d1, b2[None, :16], b2[None, 16:],
                W3[:16], W3[16:])
  a = _scatter_kernel(32)(_pack(yL, yR), idx3, dst2d)
  yL, yR = _mid(a[0, :N], a[1, :N], d0, d1, b3[None, :32], b3[None, 32:],
                W4[:32], W4[32:])
  a = _scatter_kernel(64)(_pack(yL, yR), idx3, dst2d)
  return _fin(a[0, :N], a[1, :N], d0, d1, b4[None, :64], b4[None, 64:])


def kernel(x, edge_index, W1, b1, W2, b2, W3, b3, W4, b4):
  return _run(x, edge_index, W1, b1, W2, b2, W3, b3, W4, b4)


# speedup vs baseline: 15.6071x; 1.0787x over previous
import functools

import jax
import jax.numpy as jnp
from jax import lax
from jax.experimental import pallas as pl
from jax.experimental.pallas import tpu as pltpu
from jax.experimental.pallas import tpu_sc as plsc

N = 10000
E = 320000
EC = 128
NC = 2
NT = 16
R = 2560
EPAD = R * EC
NPAD = NT * 640
SL = NPAD // NT

_mesh = plsc.VectorSubcoreMesh(core_axis_name="c", subcore_axis_name="s")
_sc_params = pltpu.CompilerParams(use_tc_tiling_on_sc=False)


def _deg_body(dst2d_hbm, degp_hbm, acc, zbuf, didx, ones):
  c = lax.axis_index("c")
  s = lax.axis_index("s")
  w = s * NC + c
  nrows = R // (NC * NT)

  def zb(i, _):
    zbuf[pl.ds(i * 16, 16)] = jnp.zeros((16,), jnp.float32)
    return 0
  lax.fori_loop(0, SL // 16, zb, 0)

  def ob(i, _):
    ones[pl.ds(i * 16, 16)] = jnp.ones((16,), jnp.float32)
    return 0
  lax.fori_loop(0, EC // 16, ob, 0)

  pltpu.sync_copy(dst2d_hbm.at[pl.ds(w * nrows, nrows), :], didx)
  pltpu.sync_copy(zbuf, acc.at[pl.ds(s * SL, SL)])
  plsc.subcore_barrier()

  def body(j, _):
    pltpu.sync_copy(ones, acc.at[didx.at[j]], add=True)
    return 0
  lax.fori_loop(0, nrows, body, 0)

  plsc.subcore_barrier()
  pltpu.sync_copy(acc.at[pl.ds(s * SL, SL)],
                  degp_hbm.at[c, 0, pl.ds(s * SL, SL)])


_deg_kernel = pl.kernel(
    _deg_body,
    out_type=jax.ShapeDtypeStruct((NC, 1, NPAD), jnp.float32),
    mesh=_mesh,
    compiler_params=_sc_params,
    scratch_types=[
        pltpu.VMEM_SHARED((NPAD,), jnp.float32),
        pltpu.VMEM((SL,), jnp.float32),
        pltpu.VMEM((R // (NC * NT), EC), jnp.int32),
        pltpu.VMEM((EC,), jnp.float32),
    ],
)


NB = 4


def _scatter_body(y2_hbm, idx3_hbm, dst2d_hbm, out_hbm,
                  acc, sidx, didx, rows, gsem, ssem):
  c = lax.axis_index("c")
  s = lax.axis_index("s")
  nrows = R // NT
  base = s * nrows

  pltpu.sync_copy(idx3_hbm.at[c, pl.ds(base, nrows), :], sidx)
  pltpu.sync_copy(dst2d_hbm.at[pl.ds(base, nrows), :], didx)
  pltpu.sync_copy(y2_hbm.at[pl.ds(c * NPAD + s * SL, SL), :],
                  acc.at[pl.ds(s * SL, SL), :])
  plsc.subcore_barrier()

  def gstart(j, q):
    pltpu.async_copy(y2_hbm.at[sidx.at[j]], rows.at[q], gsem)

  def gwait(j, q):
    pltpu.make_async_copy(y2_hbm.at[sidx.at[j]], rows.at[q], gsem).wait()

  def sstart(j, q):
    pltpu.async_copy(rows.at[q], acc.at[didx.at[j]], ssem, add=True)

  def swait(j, q):
    pltpu.make_async_copy(rows.at[q], acc.at[didx.at[j]], ssem).wait()

  for q in range(NB):
    gstart(q, q)

  def body(j2, _):
    j0 = NB * j2
    for q in range(NB):
      gwait(j0 + q, q)
      sstart(j0 + q, q)
    for q in range(NB):
      swait(j0 + q, q)

      @pl.when(j0 + q + NB < nrows)
      def _():
        gstart(j0 + q + NB, q)
    return 0
  lax.fori_loop(0, nrows // NB, body, 0)

  plsc.subcore_barrier()
  pltpu.sync_copy(acc.at[pl.ds(s * SL, SL), :],
                  out_hbm.at[c, pl.ds(s * SL, SL), :])


@functools.cache
def _scatter_kernel(dh):
  return pl.kernel(
      _scatter_body,
      out_type=jax.ShapeDtypeStruct((NC, NPAD, dh), jnp.float32),
      mesh=_mesh,
      compiler_params=_sc_params,
      scratch_types=[
          pltpu.VMEM_SHARED((NPAD, dh), jnp.float32),
          pltpu.VMEM((R // NT, EC), jnp.int32),
          pltpu.VMEM((R // NT, EC), jnp.int32),
          pltpu.VMEM((NB, EC, dh), jnp.float32),
          pltpu.SemaphoreType.DMA,
          pltpu.SemaphoreType.DMA,
      ],
  )


BR = 1000


def _mm1_body(x_ref, d0_ref, d1_ref, w_ref, o_ref):
  dinv = lax.rsqrt(d0_ref[...] + d1_ref[...] + 1.0)
  y = jnp.dot(x_ref[...], w_ref[...],
              preferred_element_type=jnp.float32) * dinv
  h = y.shape[1] // 2
  o_ref[0] = y[:, :h]
  o_ref[1] = y[:, h:]


def _mid_body(aL_ref, aR_ref, d0_ref, d1_ref, bL_ref, bR_ref, wt_ref, wb_ref,
              o_ref):
  dinv = lax.rsqrt(d0_ref[...] + d1_ref[...] + 1.0)
  hL = jnp.maximum(aL_ref[0] * dinv + bL_ref[...], 0.0)
  hR = jnp.maximum(aR_ref[0] * dinv + bR_ref[...], 0.0)
  y = (jnp.dot(hL, wt_ref[...], preferred_element_type=jnp.float32)
       + jnp.dot(hR, wb_ref[...], preferred_element_type=jnp.float32)) * dinv
  h = y.shape[1] // 2
  o_ref[0] = y[:, :h]
  o_ref[1] = y[:, h:]


def _fin_body(aL_ref, aR_ref, d0_ref, d1_ref, bL_ref, bR_ref, o_ref):
  dinv = lax.rsqrt(d0_ref[...] + d1_ref[...] + 1.0)
  o_ref[...] = jnp.concatenate(
      [aL_ref[0] * dinv + bL_ref[...], aR_ref[0] * dinv + bR_ref[...]],
      axis=1)


def _row_spec(width):
  return pl.BlockSpec((BR, width), lambda i: (i, 0))


def _half_spec(half, width):
  return pl.BlockSpec((1, BR, width), lambda i, _h=half: (_h, i, 0))


def _out3_spec(width):
  return pl.BlockSpec((2, BR, width), lambda i: (0, i, 0))


def _whole_spec(shape):
  return pl.BlockSpec(shape, lambda i: tuple(0 for _ in shape))


def _mm1(x, d0, d1, w):
  fout = w.shape[1]
  return pl.pallas_call(
      _mm1_body,
      grid=(N // BR,),
      in_specs=[_row_spec(x.shape[1]), _row_spec(1), _row_spec(1),
                _whole_spec(w.shape)],
      out_specs=_out3_spec(fout // 2),
      out_shape=jax.ShapeDtypeStruct((2, NPAD, fout // 2), jnp.float32),
  )(x, d0, d1, w)


def _mid(a, d0, d1, bL, bR, wt, wb):
  dh = a.shape[2]
  fout = wt.shape[1]
  return pl.pallas_call(
      _mid_body,
      grid=(N // BR,),
      in_specs=[_half_spec(0, dh), _half_spec(1, dh),
                _row_spec(1), _row_spec(1),
                _whole_spec(bL.shape), _whole_spec(bR.shape),
                _whole_spec(wt.shape), _whole_spec(wb.shape)],
      out_specs=_out3_spec(fout // 2),
      out_shape=jax.ShapeDtypeStruct((2, NPAD, fout // 2), jnp.float32),
  )(a, a, d0, d1, bL, bR, wt, wb)


def _fin(a, d0, d1, bL, bR):
  dh = a.shape[2]
  return pl.pallas_call(
      _fin_body,
      grid=(N // BR,),
      in_specs=[_half_spec(0, dh), _half_spec(1, dh),
                _row_spec(1), _row_spec(1),
                _whole_spec(bL.shape), _whole_spec(bR.shape)],
      out_specs=_row_spec(2 * dh),
      out_shape=jax.ShapeDtypeStruct((N, 2 * dh), jnp.float32),
  )(a, a, d0, d1, bL, bR)


@jax.jit
def _run(x, edge_index, W1, b1, W2, b2, W3, b3, W4, b4):
  src = edge_index[0]
  dst = edge_index[1]
  src_p = jnp.concatenate([src, jnp.zeros((EPAD - E,), jnp.int32)])
  dst_p = jnp.concatenate([dst, jnp.full((EPAD - E,), N, jnp.int32)])
  src2d = src_p.reshape(R, EC)
  dst2d = dst_p.reshape(R, EC)
  idx3 = jnp.stack([src2d, src2d + NPAD])

  degp = _deg_kernel(dst2d)
  d0 = degp[0, 0, :N, None]
  d1 = degp[1, 0, :N, None]

  def scatter(y3):
    dh = y3.shape[2]
    return _scatter_kernel(dh)(y3.reshape(2 * NPAD, dh), idx3, dst2d)

  a = scatter(_mm1(x, d0, d1, W1))
  a = scatter(_mid(a, d0, d1, b1[None, :32], b1[None, 32:], W2[:32], W2[32:]))
  a = scatter(_mid(a, d0, d1, b2[None, :16], b2[None, 16:], W3[:16], W3[16:]))
  a = scatter(_mid(a, d0, d1, b3[None, :32], b3[None, 32:], W4[:32], W4[32:]))
  return _fin(a, d0, d1, b4[None, :64], b4[None, 64:])


def kernel(x, edge_index, W1, b1, W2, b2, W3, b3, W4, b4):
  return _run(x, edge_index, W1, b1, W2, b2, W3, b3, W4, b4)
